# Initial kernel scaffold; baseline (speedup 1.0000x reference)
import jax
import jax.numpy as jnp
from jax.experimental import pallas as pl


def kernel(x, edge_index, edge_attr, batch, params):
    raise NotImplementedError("write your pallas kernel here")


# speedup vs baseline: 1.1000x; 1.1000x over previous
import functools
import jax
import jax.numpy as jnp
from jax.experimental import pallas as pl

N = 10000
E = 320000
G = 64
EB = 3200
ESTEPS = E // EB
NB = 2000

_INTERP = False


def _proj_body(x_ref, wr_ref, wc_ref, prow_ref, pcol_ref):
    x = x_ref[...]
    prow_ref[...] = jnp.dot(x, wr_ref[...], preferred_element_type=jnp.float32)
    pcol_ref[...] = jnp.dot(x, wc_ref[...], preferred_element_type=jnp.float32)


def _proj(x, wrT, wcT):
    return pl.pallas_call(
        _proj_body,
        grid=(N // NB,),
        in_specs=[pl.BlockSpec((NB, 128), lambda i: (i, 0)),
                  pl.BlockSpec((128, 64), lambda i: (0, 0)),
                  pl.BlockSpec((128, 192), lambda i: (0, 0))],
        out_specs=[pl.BlockSpec((NB, 64), lambda i: (i, 0)),
                   pl.BlockSpec((NB, 192), lambda i: (i, 0))],
        out_shape=(jax.ShapeDtypeStruct((N, 64), jnp.float32),
                   jax.ShapeDtypeStruct((N, 192), jnp.float32)),
        interpret=_INTERP,
    )(x, wrT, wcT)


def _edge1_body(pr_ref, pc_ref, ea_ref, g_ref,
                w1cT_ref, b1_ref, w2T_ref, b2_ref,
                v1bT_ref, c1_ref, v2T_ref, c2_ref,
                e1_ref, h_ref, acc_ref):
    i = pl.program_id(0)

    @pl.when(i == 0)
    def _():
        acc_ref[...] = jnp.zeros_like(acc_ref)

    pr = pr_ref[...]
    pc = pc_ref[...]
    ea = ea_ref[...]
    h1 = jnp.maximum(
        pr + pc[:, :64]
        + jnp.dot(ea, w1cT_ref[...], preferred_element_type=jnp.float32)
        + b1_ref[...], 0.0)
    e1 = jnp.dot(h1, w2T_ref[...], preferred_element_type=jnp.float32) + b2_ref[...]
    n1 = jnp.maximum(
        pc[:, 64:]
        + jnp.dot(e1, v1bT_ref[...], preferred_element_type=jnp.float32)
        + c1_ref[...], 0.0)
    h = jnp.dot(n1, v2T_ref[...], preferred_element_type=jnp.float32) + c2_ref[...]
    e1_ref[...] = e1
    h_ref[...] = h

    g = g_ref[0, 0, :]
    oh = (g[:, None] == jax.lax.broadcasted_iota(jnp.int32, (EB, G), 1)
          ).astype(jnp.float32)
    ge1s = jax.lax.dot_general(oh, e1, (((0,), (0,)), ((), ())),
                               preferred_element_type=jnp.float32,
                               precision=jax.lax.Precision.HIGHEST)
    cg = jnp.sum(oh, axis=0)
    cgrow = jnp.concatenate([cg, jnp.zeros((64,), jnp.float32)])[None, :]
    se = jnp.sum(e1, axis=0)[None, :]
    sq = jnp.sum(e1 * e1, axis=0)[None, :]
    acc_ref[0:G, :] += ge1s
    acc_ref[G:G + 1, :] += cgrow
    acc_ref[G + 1:G + 2, :] += se
    acc_ref[G + 2:G + 3, :] += sq


def _edge1(prg, pcg, ea, g3, w1cT, b1, w2T, b2, v1bT, c1, v2T, c2):
    blk = lambda r, c: pl.BlockSpec((EB, c), lambda i: (i, 0))
    full = lambda shape: pl.BlockSpec(shape, lambda i: tuple(0 for _ in shape))
    return pl.pallas_call(
        _edge1_body,
        grid=(ESTEPS,),
        in_specs=[blk(EB, 64), blk(EB, 192), blk(EB, 16),
                  pl.BlockSpec((1, 1, EB), lambda i: (i, 0, 0)),
                  full((16, 64)), full((1, 64)), full((64, 128)), full((1, 128)),
                  full((128, 128)), full((1, 128)), full((128, 128)), full((1, 128))],
        out_specs=[blk(EB, 128), blk(EB, 128),
                   pl.BlockSpec((72, 128), lambda i: (0, 0))],
        out_shape=(jax.ShapeDtypeStruct((E, 128), jnp.float32),
                   jax.ShapeDtypeStruct((E, 128), jnp.float32),
                   jax.ShapeDtypeStruct((72, 128), jnp.float32)),
        interpret=_INTERP,
    )(prg, pcg, ea, g3, w1cT, b1, w2T, b2, v1bT, c1, v2T, c2)


def _node1_body(s_ref, c_ref, x_ref,
                z1aT_ref, z1bT_ref, d1_ref, z2T_ref, d2_ref,
                x1_ref, st_ref):
    i = pl.program_id(0)

    @pl.when(i == 0)
    def _():
        st_ref[...] = jnp.zeros_like(st_ref)

    s = s_ref[0] + s_ref[1]
    c = c_ref[0, :, 0:1] + c_ref[1, :, 0:1]
    agg = s / jnp.maximum(c, 1.0)
    xh = jnp.maximum(
        jnp.dot(x_ref[...], z1aT_ref[...], preferred_element_type=jnp.float32)
        + jnp.dot(agg, z1bT_ref[...], preferred_element_type=jnp.float32)
        + d1_ref[...], 0.0)
    x1 = jnp.dot(xh, z2T_ref[...], preferred_element_type=jnp.float32) + d2_ref[...]
    x1_ref[...] = x1
    st_ref[0:1, :] += jnp.sum(x1, axis=0)[None, :]
    st_ref[1:2, :] += jnp.sum(x1 * x1, axis=0)[None, :]


def _node1(ssum, scnt, x, z1aT, z1bT, d1, z2T, d2):
    return pl.pallas_call(
        _node1_body,
        grid=(N // NB,),
        in_specs=[pl.BlockSpec((2, NB, 128), lambda i: (0, i, 0)),
                  pl.BlockSpec((2, NB, 8), lambda i: (0, i, 0)),
                  pl.BlockSpec((NB, 128), lambda i: (i, 0)),
                  pl.BlockSpec((128, 128), lambda i: (0, 0)),
                  pl.BlockSpec((128, 128), lambda i: (0, 0)),
                  pl.BlockSpec((1, 128), lambda i: (0, 0)),
                  pl.BlockSpec((128, 256), lambda i: (0, 0)),
                  pl.BlockSpec((1, 256), lambda i: (0, 0))],
        out_specs=[pl.BlockSpec((NB, 256), lambda i: (i, 0)),
                   pl.BlockSpec((8, 256), lambda i: (0, 0))],
        out_shape=(jax.ShapeDtypeStruct((N, 256), jnp.float32),
                   jax.ShapeDtypeStruct((8, 256), jnp.float32)),
        interpret=_INTERP,
    )(ssum, scnt, x, z1aT, z1bT, d1, z2T, d2)


def _node2_body(x1_ref, st_ref, gn_ref, bn_ref, u1aT_ref, u1bT_ref,
                qr_ref, qc_ref):
    m = st_ref[0:1, :] * (1.0 / N)
    v = st_ref[1:2, :] * (1.0 / N) - m * m
    x1b = (x1_ref[...] - m) * jax.lax.rsqrt(v + 1e-5) * gn_ref[...] + bn_ref[...]
    qr_ref[...] = jnp.dot(x1b, u1aT_ref[...], preferred_element_type=jnp.float32)
    qc_ref[...] = jnp.dot(x1b, u1bT_ref[...], preferred_element_type=jnp.float32)


def _node2(x1, st, gn, bn, u1aT, u1bT):
    return pl.pallas_call(
        _node2_body,
        grid=(N // NB,),
        in_specs=[pl.BlockSpec((NB, 256), lambda i: (i, 0)),
                  pl.BlockSpec((8, 256), lambda i: (0, 0)),
                  pl.BlockSpec((1, 256), lambda i: (0, 0)),
                  pl.BlockSpec((1, 256), lambda i: (0, 0)),
                  pl.BlockSpec((256, 128), lambda i: (0, 0)),
                  pl.BlockSpec((256, 128), lambda i: (0, 0))],
        out_specs=[pl.BlockSpec((NB, 128), lambda i: (i, 0)),
                   pl.BlockSpec((NB, 128), lambda i: (i, 0))],
        out_shape=(jax.ShapeDtypeStruct((N, 128), jnp.float32),
                   jax.ShapeDtypeStruct((N, 128), jnp.float32)),
        interpret=_INTERP,
    )(x1, st, gn, bn, u1aT, u1bT)


def _edge2_body(qr_ref, qc_ref, e1_ref, g_ref, est_ref,
                ge_ref, be_ref, u1cT_ref, e1b_ref, u2T_ref, e2b_ref,
                acc_ref):
    i = pl.program_id(0)

    @pl.when(i == 0)
    def _():
        acc_ref[...] = jnp.zeros_like(acc_ref)

    mean = est_ref[0:1, :] * (1.0 / E)
    var = est_ref[1:2, :] * (1.0 / E) - mean * mean
    a = ge_ref[...] * jax.lax.rsqrt(var + 1e-5)
    bb = be_ref[...] - mean * a
    e1bn = e1_ref[...] * a + bb
    h2 = jnp.maximum(
        qr_ref[...] + qc_ref[...]
        + jnp.dot(e1bn, u1cT_ref[...], preferred_element_type=jnp.float32)
        + e1b_ref[...], 0.0)
    e2 = jnp.dot(h2, u2T_ref[...], preferred_element_type=jnp.float32) + e2b_ref[...]
    g = g_ref[0, 0, :]
    oh = (g[:, None] == jax.lax.broadcasted_iota(jnp.int32, (EB, G), 1)
          ).astype(jnp.float32)
    acc_ref[...] += jax.lax.dot_general(oh, e2, (((0,), (0,)), ((), ())),
                                        preferred_element_type=jnp.float32,
                                        precision=jax.lax.Precision.HIGHEST)


def _edge2(qrg, qcg, e1, g3, est, ge, be, u1cT, e1b, u2T, e2b):
    blk = lambda c: pl.BlockSpec((EB, c), lambda i: (i, 0))
    full = lambda shape: pl.BlockSpec(shape, lambda i: tuple(0 for _ in shape))
    return pl.pallas_call(
        _edge2_body,
        grid=(ESTEPS,),
        in_specs=[blk(128), blk(128), blk(128),
                  pl.BlockSpec((1, 1, EB), lambda i: (i, 0, 0)),
                  full((8, 128)), full((1, 128)), full((1, 128)),
                  full((128, 128)), full((1, 128)), full((128, 128)), full((1, 128))],
        out_specs=pl.BlockSpec((G, 128), lambda i: (0, 0)),
        out_shape=jax.ShapeDtypeStruct((G, 128), jnp.float32),
        interpret=_INTERP,
    )(qrg, qcg, e1, g3, est, ge, be, u1cT, e1b, u2T, e2b)


def _glob_body(acc1_ref, acc2_ref,
               g1geT_ref, gb1_ref, g2T_ref, gb2_ref,
               h1uT_ref, h1geT_ref, hb1_ref, h2T_ref, hb2_ref,
               out_ref):
    inv = 1.0 / jnp.maximum(acc1_ref[G:G + 1, 0:G], 1.0)
    eye = (jax.lax.broadcasted_iota(jnp.int32, (G, G), 0)
           == jax.lax.broadcasted_iota(jnp.int32, (G, G), 1)).astype(jnp.float32)
    dinv = eye * inv
    ge1 = jnp.dot(dinv, acc1_ref[0:G, :], preferred_element_type=jnp.float32,
                  precision=jax.lax.Precision.HIGHEST)
    u1h = jnp.maximum(
        jnp.dot(ge1, g1geT_ref[...], preferred_element_type=jnp.float32)
        + gb1_ref[...], 0.0)
    u1 = jnp.dot(u1h, g2T_ref[...], preferred_element_type=jnp.float32) + gb2_ref[...]
    ge2 = jnp.dot(dinv, acc2_ref[...], preferred_element_type=jnp.float32,
                  precision=jax.lax.Precision.HIGHEST)
    u2h = jnp.maximum(
        jnp.dot(u1, h1uT_ref[...], preferred_element_type=jnp.float32)
        + jnp.dot(ge2, h1geT_ref[...], preferred_element_type=jnp.float32)
        + hb1_ref[...], 0.0)
    out_ref[...] = (jnp.dot(u2h, h2T_ref[...], preferred_element_type=jnp.float32)
                    + hb2_ref[...])


def _glob(acc1, acc2, g1geT, gb1, g2T, gb2, h1uT, h1geT, hb1, h2T, hb2):
    return pl.pallas_call(
        _glob_body,
        out_shape=jax.ShapeDtypeStruct((G, 1), jnp.float32),
        interpret=_INTERP,
    )(acc1, acc2, g1geT, gb1, g2T, gb2, h1uT, h1geT, hb1, h2T, hb2)


def kernel(x, edge_index, edge_attr, batch, params):
    row = edge_index[0]
    col = edge_index[1]
    p1 = params['l1']
    (W1, b1), (W2, b2) = p1['edge']
    (V1, c1), (V2, c2) = p1['node1']
    (Z1, d1), (Z2, d2) = p1['node2']
    (Gw1, gb1), (Gw2, gb2) = p1['glob']
    gme, gbe = params['bn_edge']
    gmn, gbn = params['bn_node']
    p2 = params['l2']
    (U1, e1b), (U2, e2b) = p2['edge']
    (Hw1, hb1), (Hw2, hb2) = p2['glob']

    r2 = lambda v: v[None, :]

    wrT = W1[:, :128].T
    wcT = jnp.concatenate([W1[:, 128:256].T, V1[:, :128].T], axis=1)
    prow, pcol = _proj(x, wrT, wcT)

    prg = prow[row]
    pcg = pcol[col]
    g = batch[row]
    g3 = g.reshape(ESTEPS, 1, EB)

    e1, h, acc1 = _edge1(prg, pcg, edge_attr, g3,
                         W1[:, 256:].T, r2(b1), W2.T, r2(b2),
                         V1[:, 128:].T, r2(c1), V2.T, r2(c2))

    s = jax.ops.segment_sum(h, row, num_segments=N)
    cnt = jax.ops.segment_sum(jnp.ones((E,), jnp.float32), row, num_segments=N)
    ssum = jnp.stack([s, jnp.zeros_like(s)])
    scnt = jnp.zeros((2, N, 8), jnp.float32).at[0, :, 0].set(cnt)

    x1, xst = _node1(ssum, scnt, x,
                     Z1[:, :128].T, Z1[:, 128:].T, r2(d1), Z2.T, r2(d2))
    qrow, qcol = _node2(x1, xst, r2(gmn), r2(gbn),
                        U1[:, :256].T, U1[:, 256:512].T)

    qrg = qrow[row]
    qcg = qcol[col]

    est = jnp.zeros((8, 128), jnp.float32).at[0:2, :].set(acc1[G + 1:G + 3, :])
    acc2 = _edge2(qrg, qcg, e1, g3, est,
                  r2(gme), r2(gbe), U1[:, 512:].T, r2(e1b), U2.T, r2(e2b))

    return _glob(acc1, acc2, Gw1[:, 1:].T, r2(gb1), Gw2.T, r2(gb2),
                 Hw1[:, :1].T, Hw1[:, 1:].T, r2(hb1), Hw2.T, r2(hb2))


# speedup vs baseline: 4.6589x; 4.2355x over previous
import functools
import jax
import jax.numpy as jnp
from jax import lax
from jax.experimental import pallas as pl
from jax.experimental.pallas import tpu as pltpu
from jax.experimental.pallas import tpu_sc as plsc

N = 10000
E = 320000
G = 64
EB = 2560
NB = 2000

NW = 32
CB = 80
NPAD = 10240
NPC = NPAD // 16

CH_NCH = (64, 61)
CH_OFF = (0, NW * CH_NCH[0] * CB)

_INTERP = False


def _proj_body(x_ref, b_ref, wr_ref, wc_ref, tr_ref, tc_ref):
    x = x_ref[...]
    p1 = jnp.dot(x, wr_ref[...], preferred_element_type=jnp.float32)
    bf = b_ref[...].astype(jnp.float32)
    pad = jnp.zeros((p1.shape[0], 63), jnp.float32)
    tr_ref[...] = jnp.concatenate([p1, bf, pad], axis=1)
    tc_ref[...] = jnp.dot(x, wc_ref[...], preferred_element_type=jnp.float32)


def _proj(x, batch1, wrT, wcTp):
    return pl.pallas_call(
        _proj_body,
        grid=(N // NB,),
        in_specs=[pl.BlockSpec((NB, 128), lambda i: (i, 0)),
                  pl.BlockSpec((NB, 1), lambda i: (i, 0)),
                  pl.BlockSpec((128, 64), lambda i: (0, 0)),
                  pl.BlockSpec((128, 256), lambda i: (0, 0))],
        out_specs=[pl.BlockSpec((NB, 128), lambda i: (i, 0)),
                   pl.BlockSpec((NB, 256), lambda i: (i, 0))],
        out_shape=(jax.ShapeDtypeStruct((N, 128), jnp.float32),
                   jax.ShapeDtypeStruct((N, 256), jnp.float32)),
        interpret=_INTERP,
    )(x, batch1, wrT, wcTp)


def _sc_gather(tabs, widths, row_rs, col_rs, nch):
    epw = nch * CB
    EC = NW * epw
    mesh = plsc.VectorSubcoreMesh(core_axis_name="c", subcore_axis_name="s")
    wr, wc = widths

    @functools.partial(
        pl.kernel, mesh=mesh,
        out_type=(jax.ShapeDtypeStruct((EC, wr), jnp.float32),
                  jax.ShapeDtypeStruct((EC, wc), jnp.float32)),
        scratch_types=[
            pltpu.VMEM((nch, CB), jnp.int32),
            pltpu.VMEM((nch, CB), jnp.int32),
            pltpu.VMEM((CB, wr), jnp.float32),
            pltpu.VMEM((CB, wr), jnp.float32),
            pltpu.VMEM((CB, wc), jnp.float32),
            pltpu.VMEM((CB, wc), jnp.float32),
            pltpu.SemaphoreType.DMA,
            pltpu.SemaphoreType.DMA,
            pltpu.SemaphoreType.DMA,
            pltpu.SemaphoreType.DMA,
        ],
    )
    def k(rt_h, ct_h, row_rs_h, col_rs_h,
          ro_o, co_o,
          ridx2, cidx2, br0, br1, bc0, bc1,
          sa, sb, sc, sd):
        wid = lax.axis_index("s") * 2 + lax.axis_index("c")
        base = wid * epw
        pltpu.sync_copy(row_rs_h.at[wid], ridx2)
        pltpu.sync_copy(col_rs_h.at[wid], cidx2)

        def pair(k2, _):
            i0 = k2 * 2
            i1 = i0 + 1
            d0 = pltpu.async_copy(rt_h.at[ridx2.at[i0]], br0, sa)
            d1 = pltpu.async_copy(ct_h.at[cidx2.at[i0]], bc0, sb)
            d2 = pltpu.async_copy(rt_h.at[ridx2.at[i1]], br1, sc)
            d3 = pltpu.async_copy(ct_h.at[cidx2.at[i1]], bc1, sd)
            d0.wait()
            o0 = pltpu.async_copy(br0, ro_o.at[pl.ds(base + i0 * CB, CB)], sa)
            d1.wait()
            o1 = pltpu.async_copy(bc0, co_o.at[pl.ds(base + i0 * CB, CB)], sb)
            d2.wait()
            o2 = pltpu.async_copy(br1, ro_o.at[pl.ds(base + i1 * CB, CB)], sc)
            d3.wait()
            o3 = pltpu.async_copy(bc1, co_o.at[pl.ds(base + i1 * CB, CB)], sd)
            o0.wait(); o1.wait(); o2.wait(); o3.wait()
            return 0

        lax.fori_loop(0, nch // 2, pair, 0, unroll=False)
        if nch % 2:
            iL = nch - 1
            dL0 = pltpu.async_copy(rt_h.at[ridx2.at[iL]], br0, sa)
            dL1 = pltpu.async_copy(ct_h.at[cidx2.at[iL]], bc0, sb)
            dL0.wait()
            pltpu.sync_copy(br0, ro_o.at[pl.ds(base + iL * CB, CB)])
            dL1.wait()
            pltpu.sync_copy(bc0, co_o.at[pl.ds(base + iL * CB, CB)])

    return k(tabs[0], tabs[1], row_rs, col_rs)


def _sc_scatter(h, row_rs, zsum, nch):
    epw = nch * CB
    mesh = plsc.VectorSubcoreMesh(core_axis_name="c", subcore_axis_name="s")

    @functools.partial(
        pl.kernel, mesh=mesh,
        out_type=jax.ShapeDtypeStruct((2, NPAD, 128), jnp.float32),
        scratch_types=[
            pltpu.VMEM((nch, CB), jnp.int32),
            pltpu.VMEM((CB, 128), jnp.float32),
            pltpu.VMEM((CB, 128), jnp.float32),
            pltpu.VMEM_SHARED((NPAD, 128), jnp.float32),
            pltpu.SemaphoreType.DMA,
            pltpu.SemaphoreType.DMA,
        ],
    )
    def k(h_h, row_rs_h, zsum_h,
          ssum_o,
          ridx2, hb0, hb1, acc, sa, sb):
        cid = lax.axis_index("c")
        sid = lax.axis_index("s")
        wid = sid * 2 + cid
        base = wid * epw
        pltpu.sync_copy(row_rs_h.at[wid], ridx2)

        @pl.when(sid == 0)
        def _():
            pltpu.sync_copy(zsum_h, acc)

        plsc.subcore_barrier()

        def pair(k2, _):
            i0 = k2 * 2
            i1 = i0 + 1
            d0 = pltpu.async_copy(h_h.at[pl.ds(base + i0 * CB, CB)], hb0, sa)
            d1 = pltpu.async_copy(h_h.at[pl.ds(base + i1 * CB, CB)], hb1, sb)
            d0.wait()
            pltpu.sync_copy(hb0, acc.at[ridx2.at[i0]], add=True)
            d1.wait()
            pltpu.sync_copy(hb1, acc.at[ridx2.at[i1]], add=True)
            return 0

        lax.fori_loop(0, nch // 2, pair, 0, unroll=False)
        if nch % 2:
            iL = nch - 1
            dL = pltpu.async_copy(h_h.at[pl.ds(base + iL * CB, CB)], hb0, sa)
            dL.wait()
            pltpu.sync_copy(hb0, acc.at[ridx2.at[iL]], add=True)
        plsc.subcore_barrier()
        pltpu.sync_copy(acc.at[pl.ds(sid * NPC, NPC)],
                        ssum_o.at[cid, pl.ds(sid * NPC, NPC)])

    return k(h, row_rs, zsum)


def _sc_count(row_rs, zcnt, cnt_src):
    nch = E // NW // CB
    mesh = plsc.VectorSubcoreMesh(core_axis_name="c", subcore_axis_name="s")

    @functools.partial(
        pl.kernel, mesh=mesh,
        out_type=jax.ShapeDtypeStruct((2, NPAD, 128), jnp.float32),
        scratch_types=[
            pltpu.VMEM((nch, CB), jnp.int32),
            pltpu.VMEM((CB, 128), jnp.float32),
            pltpu.VMEM_SHARED((NPAD, 128), jnp.float32),
        ],
    )
    def k(row_rs_h, zcnt_h, csrc_h, scnt_o, ridx2, csrc, accc):
        cid = lax.axis_index("c")
        sid = lax.axis_index("s")
        wid = sid * 2 + cid
        pltpu.sync_copy(row_rs_h.at[wid], ridx2)
        pltpu.sync_copy(csrc_h, csrc)

        @pl.when(sid == 0)
        def _():
            pltpu.sync_copy(zcnt_h, accc)

        plsc.subcore_barrier()

        def step(i, _):
            pltpu.sync_copy(csrc, accc.at[ridx2.at[i]], add=True)
            return 0

        lax.fori_loop(0, nch, step, 0, unroll=False)
        plsc.subcore_barrier()
        pltpu.sync_copy(accc.at[pl.ds(sid * NPC, NPC)],
                        scnt_o.at[cid, pl.ds(sid * NPC, NPC)])

    return k(row_rs, zcnt, cnt_src)


def _edge1_body(pr_ref, pc_ref, ea_ref,
                w1cT_ref, b1_ref, w2T_ref, b2_ref,
                v1bT_ref, c1_ref, v2T_ref, c2_ref,
                e1_ref, h_ref, acc_ref):
    i = pl.program_id(0)

    @pl.when(i == 0)
    def _():
        acc_ref[...] = jnp.zeros_like(acc_ref)

    pr = pr_ref[:, 0:64]
    pc = pc_ref[...]
    ea = ea_ref[...]
    h1 = jnp.maximum(
        pr + pc[:, :64]
        + jnp.dot(ea, w1cT_ref[...], preferred_element_type=jnp.float32)
        + b1_ref[...], 0.0)
    e1 = jnp.dot(h1, w2T_ref[...], preferred_element_type=jnp.float32) + b2_ref[...]
    n1 = jnp.maximum(
        pc[:, 64:192]
        + jnp.dot(e1, v1bT_ref[...], preferred_element_type=jnp.float32)
        + c1_ref[...], 0.0)
    h = jnp.dot(n1, v2T_ref[...], preferred_element_type=jnp.float32) + c2_ref[...]
    e1_ref[...] = e1
    h_ref[...] = h

    gi = pr_ref[:, 64:65].astype(jnp.int32)
    oh = (gi == jax.lax.broadcasted_iota(jnp.int32, (EB, G), 1)
          ).astype(jnp.float32)
    ge1s = jax.lax.dot_general(oh, e1, (((0,), (0,)), ((), ())),
                               preferred_element_type=jnp.float32,
                               precision=jax.lax.Precision.HIGHEST)
    cg = jnp.sum(oh, axis=0)
    cgrow = jnp.concatenate([cg, jnp.zeros((64,), jnp.float32)])[None, :]
    se = jnp.sum(e1, axis=0)[None, :]
    sq = jnp.sum(e1 * e1, axis=0)[None, :]
    acc_ref[0:G, :] += ge1s
    acc_ref[G:G + 1, :] += cgrow
    acc_ref[G + 1:G + 2, :] += se
    acc_ref[G + 2:G + 3, :] += sq


def _edge1(prg, pcg, ea, w1cT, b1, w2T, b2, v1bT, c1, v2T, c2):
    EC = prg.shape[0]
    blk = lambda r, c: pl.BlockSpec((EB, c), lambda i: (i, 0))
    full = lambda shape: pl.BlockSpec(shape, lambda i: tuple(0 for _ in shape))
    return pl.pallas_call(
        _edge1_body,
        grid=(EC // EB,),
        in_specs=[blk(EB, 128), blk(EB, 256), blk(EB, 16),
                  full((16, 64)), full((1, 64)), full((64, 128)), full((1, 128)),
                  full((128, 128)), full((1, 128)), full((128, 128)), full((1, 128))],
        out_specs=[blk(EB, 128), blk(EB, 128),
                   pl.BlockSpec((72, 128), lambda i: (0, 0))],
        out_shape=(jax.ShapeDtypeStruct((EC, 128), jnp.float32),
                   jax.ShapeDtypeStruct((EC, 128), jnp.float32),
                   jax.ShapeDtypeStruct((72, 128), jnp.float32)),
        interpret=_INTERP,
    )(prg, pcg, ea, w1cT, b1, w2T, b2, v1bT, c1, v2T, c2)


def _node1_body(sa_ref, sb_ref, c_ref, x_ref,
                z1aT_ref, z1bT_ref, d1_ref, z2T_ref, d2_ref,
                x1_ref, st_ref):
    i = pl.program_id(0)

    @pl.when(i == 0)
    def _():
        st_ref[...] = jnp.zeros_like(st_ref)

    s = sa_ref[0] + sa_ref[1] + sb_ref[0] + sb_ref[1]
    c = c_ref[0, :, 0:1] + c_ref[1, :, 0:1]
    agg = s / jnp.maximum(c, 1.0)
    xh = jnp.maximum(
        jnp.dot(x_ref[...], z1aT_ref[...], preferred_element_type=jnp.float32)
        + jnp.dot(agg, z1bT_ref[...], preferred_element_type=jnp.float32)
        + d1_ref[...], 0.0)
    x1 = jnp.dot(xh, z2T_ref[...], preferred_element_type=jnp.float32) + d2_ref[...]
    x1_ref[...] = x1
    st_ref[0:1, :] += jnp.sum(x1, axis=0)[None, :]
    st_ref[1:2, :] += jnp.sum(x1 * x1, axis=0)[None, :]


def _node1(ssA, ssB, scnt, x, z1aT, z1bT, d1, z2T, d2):
    return pl.pallas_call(
        _node1_body,
        grid=(N // NB,),
        in_specs=[pl.BlockSpec((2, NB, 128), lambda i: (0, i, 0)),
                  pl.BlockSpec((2, NB, 128), lambda i: (0, i, 0)),
                  pl.BlockSpec((2, NB, 128), lambda i: (0, i, 0)),
                  pl.BlockSpec((NB, 128), lambda i: (i, 0)),
                  pl.BlockSpec((128, 128), lambda i: (0, 0)),
                  pl.BlockSpec((128, 128), lambda i: (0, 0)),
                  pl.BlockSpec((1, 128), lambda i: (0, 0)),
                  pl.BlockSpec((128, 256), lambda i: (0, 0)),
                  pl.BlockSpec((1, 256), lambda i: (0, 0))],
        out_specs=[pl.BlockSpec((NB, 256), lambda i: (i, 0)),
                   pl.BlockSpec((8, 256), lambda i: (0, 0))],
        out_shape=(jax.ShapeDtypeStruct((N, 256), jnp.float32),
                   jax.ShapeDtypeStruct((8, 256), jnp.float32)),
        interpret=_INTERP,
    )(ssA, ssB, scnt, x, z1aT, z1bT, d1, z2T, d2)


def _node2_body(x1_ref, st_ref, gn_ref, bn_ref, u1aT_ref, u1bT_ref,
                qr_ref, qc_ref):
    m = st_ref[0:1, :] * (1.0 / N)
    v = st_ref[1:2, :] * (1.0 / N) - m * m
    x1b = (x1_ref[...] - m) * jax.lax.rsqrt(v + 1e-5) * gn_ref[...] + bn_ref[...]
    qr_ref[...] = jnp.dot(x1b, u1aT_ref[...], preferred_element_type=jnp.float32)
    qc_ref[...] = jnp.dot(x1b, u1bT_ref[...], preferred_element_type=jnp.float32)


def _node2(x1, st, gn, bn, u1aT, u1bT):
    return pl.pallas_call(
        _node2_body,
        grid=(N // NB,),
        in_specs=[pl.BlockSpec((NB, 256), lambda i: (i, 0)),
                  pl.BlockSpec((8, 256), lambda i: (0, 0)),
                  pl.BlockSpec((1, 256), lambda i: (0, 0)),
                  pl.BlockSpec((1, 256), lambda i: (0, 0)),
                  pl.BlockSpec((256, 128), lambda i: (0, 0)),
                  pl.BlockSpec((256, 128), lambda i: (0, 0))],
        out_specs=[pl.BlockSpec((NB, 128), lambda i: (i, 0)),
                   pl.BlockSpec((NB, 128), lambda i: (i, 0))],
        out_shape=(jax.ShapeDtypeStruct((N, 128), jnp.float32),
                   jax.ShapeDtypeStruct((N, 128), jnp.float32)),
        interpret=_INTERP,
    )(x1, st, gn, bn, u1aT, u1bT)


def _edge2_body(qr_ref, qc_ref, e1_ref, g_ref, est_ref,
                ge_ref, be_ref, u1cT_ref, e1b_ref, u2T_ref, e2b_ref,
                acc_ref):
    i = pl.program_id(0)

    @pl.when(i == 0)
    def _():
        acc_ref[...] = jnp.zeros_like(acc_ref)

    mean = est_ref[0:1, :] * (1.0 / E)
    var = est_ref[1:2, :] * (1.0 / E) - mean * mean
    a = ge_ref[...] * jax.lax.rsqrt(var + 1e-5)
    bb = be_ref[...] - mean * a
    e1bn = e1_ref[...] * a + bb
    h2 = jnp.maximum(
        qr_ref[...] + qc_ref[...]
        + jnp.dot(e1bn, u1cT_ref[...], preferred_element_type=jnp.float32)
        + e1b_ref[...], 0.0)
    e2 = jnp.dot(h2, u2T_ref[...], preferred_element_type=jnp.float32) + e2b_ref[...]
    gi = g_ref[:, 64:65].astype(jnp.int32)
    oh = (gi == jax.lax.broadcasted_iota(jnp.int32, (EB, G), 1)
          ).astype(jnp.float32)
    acc_ref[...] += jax.lax.dot_general(oh, e2, (((0,), (0,)), ((), ())),
                                        preferred_element_type=jnp.float32,
                                        precision=jax.lax.Precision.HIGHEST)


def _edge2(qrg, qcg, e1, gs, est, ge, be, u1cT, e1b, u2T, e2b):
    EC = qrg.shape[0]
    blk = lambda c: pl.BlockSpec((EB, c), lambda i: (i, 0))
    full = lambda shape: pl.BlockSpec(shape, lambda i: tuple(0 for _ in shape))
    return pl.pallas_call(
        _edge2_body,
        grid=(EC // EB,),
        in_specs=[blk(128), blk(128), blk(128),
                  pl.BlockSpec((EB, 128), lambda i: (i, 0)),
                  full((8, 128)), full((1, 128)), full((1, 128)),
                  full((128, 128)), full((1, 128)), full((128, 128)), full((1, 128))],
        out_specs=pl.BlockSpec((G, 128), lambda i: (0, 0)),
        out_shape=jax.ShapeDtypeStruct((G, 128), jnp.float32),
        interpret=_INTERP,
    )(qrg, qcg, e1, gs, est, ge, be, u1cT, e1b, u2T, e2b)


def _glob_body(acc1_ref, acc2_ref,
               g1geT_ref, gb1_ref, g2T_ref, gb2_ref,
               h1uT_ref, h1geT_ref, hb1_ref, h2T_ref, hb2_ref,
               out_ref):
    inv = 1.0 / jnp.maximum(acc1_ref[G:G + 1, 0:G], 1.0)
    eye = (jax.lax.broadcasted_iota(jnp.int32, (G, G), 0)
           == jax.lax.broadcasted_iota(jnp.int32, (G, G), 1)).astype(jnp.float32)
    dinv = eye * inv
    ge1 = jnp.dot(dinv, acc1_ref[0:G, :], preferred_element_type=jnp.float32,
                  precision=jax.lax.Precision.HIGHEST)
    u1h = jnp.maximum(
        jnp.dot(ge1, g1geT_ref[...], preferred_element_type=jnp.float32)
        + gb1_ref[...], 0.0)
    u1 = jnp.dot(u1h, g2T_ref[...], preferred_element_type=jnp.float32) + gb2_ref[...]
    ge2 = jnp.dot(dinv, acc2_ref[...], preferred_element_type=jnp.float32,
                  precision=jax.lax.Precision.HIGHEST)
    u2h = jnp.maximum(
        jnp.dot(u1, h1uT_ref[...], preferred_element_type=jnp.float32)
        + jnp.dot(ge2, h1geT_ref[...], preferred_element_type=jnp.float32)
        + hb1_ref[...], 0.0)
    out_ref[...] = (jnp.dot(u2h, h2T_ref[...], preferred_element_type=jnp.float32)
                    + hb2_ref[...])


def _glob(acc1, acc2, g1geT, gb1, g2T, gb2, h1uT, h1geT, hb1, h2T, hb2):
    return pl.pallas_call(
        _glob_body,
        out_shape=jax.ShapeDtypeStruct((G, 1), jnp.float32),
        interpret=_INTERP,
    )(acc1, acc2, g1geT, gb1, g2T, gb2, h1uT, h1geT, hb1, h2T, hb2)


def kernel(x, edge_index, edge_attr, batch, params):
    row = edge_index[0]
    col = edge_index[1]
    p1 = params['l1']
    (W1, b1), (W2, b2) = p1['edge']
    (V1, c1), (V2, c2) = p1['node1']
    (Z1, d1), (Z2, d2) = p1['node2']
    (Gw1, gb1), (Gw2, gb2) = p1['glob']
    gme, gbe = params['bn_edge']
    gmn, gbn = params['bn_node']
    p2 = params['l2']
    (U1, e1b), (U2, e2b) = p2['edge']
    (Hw1, hb1), (Hw2, hb2) = p2['glob']

    r2 = lambda v: v[None, :]

    wrT = W1[:, :128].T
    wcTp = jnp.concatenate([W1[:, 128:256].T, V1[:, :128].T,
                            jnp.zeros((128, 64), jnp.float32)], axis=1)
    tr, tcol = _proj(x, batch[:, None], wrT, wcTp)

    def chunk_idx(a, ci):
        nch = CH_NCH[ci]
        off = CH_OFF[ci]
        return a[off:off + NW * nch * CB].reshape(NW, nch, CB)

    row_c = [chunk_idx(row, 0), chunk_idx(row, 1)]
    col_c = [chunk_idx(col, 0), chunk_idx(col, 1)]
    ea_c = [edge_attr[CH_OFF[0]:CH_OFF[1]], edge_attr[CH_OFF[1]:]]

    zsum = jnp.zeros((NPAD, 128), jnp.float32)
    cnt_src = jnp.zeros((CB, 128), jnp.float32).at[:, 0].set(1.0)
    row_rs_full = row.reshape(NW, E // NW // CB, CB)
    scnt = _sc_count(row_rs_full, zsum, cnt_src)

    we = (W1[:, 256:].T, r2(b1), W2.T, r2(b2), V1[:, 128:].T, r2(c1), V2.T, r2(c2))
    pg = [_sc_gather((tr, tcol), (128, 256), row_c[i], col_c[i], CH_NCH[i])
          for i in range(2)]
    eo = [_edge1(pg[i][0], pg[i][1], ea_c[i], *we) for i in range(2)]
    ss = [_sc_scatter(eo[i][1], row_c[i], zsum, CH_NCH[i]) for i in range(2)]
    acc1 = eo[0][2] + eo[1][2]

    x1, xst = _node1(ss[0], ss[1], scnt, x,
                     Z1[:, :128].T, Z1[:, 128:].T, r2(d1), Z2.T, r2(d2))
    qrow, qcol = _node2(x1, xst, r2(gmn), r2(gbn),
                        U1[:, :256].T, U1[:, 256:512].T)

    est = jnp.zeros((8, 128), jnp.float32).at[0:2, :].set(acc1[G + 1:G + 3, :])
    w2e = (r2(gme), r2(gbe), U1[:, 512:].T, r2(e1b), U2.T, r2(e2b))
    qg = [_sc_gather((qrow, qcol), (128, 128), row_c[i], col_c[i], CH_NCH[i])
          for i in range(2)]
    acc2 = sum(_edge2(qg[i][0], qg[i][1], eo[i][0], pg[i][0], est, *w2e)
               for i in range(2))

    return _glob(acc1, acc2, Gw1[:, 1:].T, r2(gb1), Gw2.T, r2(gb2),
                 Hw1[:, :1].T, Hw1[:, 1:].T, r2(hb1), Hw2.T, r2(hb2))


# speedup vs baseline: 4.7061x; 1.0101x over previous
import functools
import jax
import jax.numpy as jnp
from jax import lax
from jax.experimental import pallas as pl
from jax.experimental.pallas import tpu as pltpu
from jax.experimental.pallas import tpu_sc as plsc

N = 10000
E = 320000
G = 64
EB = 2560
NB = 2000

NW = 32
CB = 80
NPAD = 10240
NPC = NPAD // 16

CH_NCH = (42, 42, 41)
CH_OFF = (0, NW * 42 * CB, NW * 84 * CB)
NCHUNK = len(CH_NCH)

_INTERP = False


def _proj_body(x_ref, b_ref, wr_ref, wc_ref, tr_ref, tc_ref):
    x = x_ref[...]
    p1 = jnp.dot(x, wr_ref[...], preferred_element_type=jnp.float32)
    bf = b_ref[...].astype(jnp.float32)
    pad = jnp.zeros((p1.shape[0], 63), jnp.float32)
    tr_ref[...] = jnp.concatenate([p1, bf, pad], axis=1)
    tc_ref[...] = jnp.dot(x, wc_ref[...], preferred_element_type=jnp.float32)


def _proj(x, batch1, wrT, wcTp):
    return pl.pallas_call(
        _proj_body,
        grid=(N // NB,),
        in_specs=[pl.BlockSpec((NB, 128), lambda i: (i, 0)),
                  pl.BlockSpec((NB, 1), lambda i: (i, 0)),
                  pl.BlockSpec((128, 64), lambda i: (0, 0)),
                  pl.BlockSpec((128, 256), lambda i: (0, 0))],
        out_specs=[pl.BlockSpec((NB, 128), lambda i: (i, 0)),
                   pl.BlockSpec((NB, 256), lambda i: (i, 0))],
        out_shape=(jax.ShapeDtypeStruct((N, 128), jnp.float32),
                   jax.ShapeDtypeStruct((N, 256), jnp.float32)),
        interpret=_INTERP,
    )(x, batch1, wrT, wcTp)


def _sc_gather(tabs, widths, row_rs, col_rs, nch):
    epw = nch * CB
    EC = NW * epw
    mesh = plsc.VectorSubcoreMesh(core_axis_name="c", subcore_axis_name="s")
    wr, wc = widths

    @functools.partial(
        pl.kernel, mesh=mesh,
        out_type=(jax.ShapeDtypeStruct((EC, wr), jnp.float32),
                  jax.ShapeDtypeStruct((EC, wc), jnp.float32)),
        scratch_types=[
            pltpu.VMEM((nch, CB), jnp.int32),
            pltpu.VMEM((nch, CB), jnp.int32),
            pltpu.VMEM((CB, wr), jnp.float32),
            pltpu.VMEM((CB, wr), jnp.float32),
            pltpu.VMEM((CB, wc), jnp.float32),
            pltpu.VMEM((CB, wc), jnp.float32),
            pltpu.SemaphoreType.DMA,
            pltpu.SemaphoreType.DMA,
            pltpu.SemaphoreType.DMA,
            pltpu.SemaphoreType.DMA,
        ],
    )
    def k(rt_h, ct_h, row_rs_h, col_rs_h,
          ro_o, co_o,
          ridx2, cidx2, br0, br1, bc0, bc1,
          sa, sb, sc, sd):
        wid = lax.axis_index("s") * 2 + lax.axis_index("c")
        base = wid * epw
        pltpu.sync_copy(row_rs_h.at[wid], ridx2)
        pltpu.sync_copy(col_rs_h.at[wid], cidx2)

        def pair(k2, _):
            i0 = k2 * 2
            i1 = i0 + 1
            d0 = pltpu.async_copy(rt_h.at[ridx2.at[i0]], br0, sa)
            d1 = pltpu.async_copy(ct_h.at[cidx2.at[i0]], bc0, sb)
            d2 = pltpu.async_copy(rt_h.at[ridx2.at[i1]], br1, sc)
            d3 = pltpu.async_copy(ct_h.at[cidx2.at[i1]], bc1, sd)
            d0.wait()
            o0 = pltpu.async_copy(br0, ro_o.at[pl.ds(base + i0 * CB, CB)], sa)
            d1.wait()
            o1 = pltpu.async_copy(bc0, co_o.at[pl.ds(base + i0 * CB, CB)], sb)
            d2.wait()
            o2 = pltpu.async_copy(br1, ro_o.at[pl.ds(base + i1 * CB, CB)], sc)
            d3.wait()
            o3 = pltpu.async_copy(bc1, co_o.at[pl.ds(base + i1 * CB, CB)], sd)
            o0.wait(); o1.wait(); o2.wait(); o3.wait()
            return 0

        lax.fori_loop(0, nch // 2, pair, 0, unroll=False)
        if nch % 2:
            iL = nch - 1
            dL0 = pltpu.async_copy(rt_h.at[ridx2.at[iL]], br0, sa)
            dL1 = pltpu.async_copy(ct_h.at[cidx2.at[iL]], bc0, sb)
            dL0.wait()
            pltpu.sync_copy(br0, ro_o.at[pl.ds(base + iL * CB, CB)])
            dL1.wait()
            pltpu.sync_copy(bc0, co_o.at[pl.ds(base + iL * CB, CB)])

    return k(tabs[0], tabs[1], row_rs, col_rs)


def _sc_scatter(h, row_rs, zsum, nch):
    epw = nch * CB
    mesh = plsc.VectorSubcoreMesh(core_axis_name="c", subcore_axis_name="s")

    @functools.partial(
        pl.kernel, mesh=mesh,
        out_type=jax.ShapeDtypeStruct((2, NPAD, 128), jnp.float32),
        scratch_types=[
            pltpu.VMEM((nch, CB), jnp.int32),
            pltpu.VMEM((CB, 128), jnp.float32),
            pltpu.VMEM((CB, 128), jnp.float32),
            pltpu.VMEM_SHARED((NPAD, 128), jnp.float32),
            pltpu.SemaphoreType.DMA,
            pltpu.SemaphoreType.DMA,
        ],
    )
    def k(h_h, row_rs_h, zsum_h,
          ssum_o,
          ridx2, hb0, hb1, acc, sa, sb):
        cid = lax.axis_index("c")
        sid = lax.axis_index("s")
        wid = sid * 2 + cid
        base = wid * epw
        pltpu.sync_copy(row_rs_h.at[wid], ridx2)

        @pl.when(sid == 0)
        def _():
            pltpu.sync_copy(zsum_h, acc)

        plsc.subcore_barrier()

        def pair(k2, _):
            i0 = k2 * 2
            i1 = i0 + 1
            d0 = pltpu.async_copy(h_h.at[pl.ds(base + i0 * CB, CB)], hb0, sa)
            d1 = pltpu.async_copy(h_h.at[pl.ds(base + i1 * CB, CB)], hb1, sb)
            d0.wait()
            pltpu.sync_copy(hb0, acc.at[ridx2.at[i0]], add=True)
            d1.wait()
            pltpu.sync_copy(hb1, acc.at[ridx2.at[i1]], add=True)
            return 0

        lax.fori_loop(0, nch // 2, pair, 0, unroll=False)
        if nch % 2:
            iL = nch - 1
            dL = pltpu.async_copy(h_h.at[pl.ds(base + iL * CB, CB)], hb0, sa)
            dL.wait()
            pltpu.sync_copy(hb0, acc.at[ridx2.at[iL]], add=True)
        plsc.subcore_barrier()
        pltpu.sync_copy(acc.at[pl.ds(sid * NPC, NPC)],
                        ssum_o.at[cid, pl.ds(sid * NPC, NPC)])

    return k(h, row_rs, zsum)


def _sc_count(row_rs, zcnt, cnt_src):
    nch = E // NW // CB
    mesh = plsc.VectorSubcoreMesh(core_axis_name="c", subcore_axis_name="s")

    @functools.partial(
        pl.kernel, mesh=mesh,
        out_type=jax.ShapeDtypeStruct((2, NPAD, 128), jnp.float32),
        scratch_types=[
            pltpu.VMEM((nch, CB), jnp.int32),
            pltpu.VMEM((CB, 128), jnp.float32),
            pltpu.VMEM_SHARED((NPAD, 128), jnp.float32),
        ],
    )
    def k(row_rs_h, zcnt_h, csrc_h, scnt_o, ridx2, csrc, accc):
        cid = lax.axis_index("c")
        sid = lax.axis_index("s")
        wid = sid * 2 + cid
        pltpu.sync_copy(row_rs_h.at[wid], ridx2)
        pltpu.sync_copy(csrc_h, csrc)

        @pl.when(sid == 0)
        def _():
            pltpu.sync_copy(zcnt_h, accc)

        plsc.subcore_barrier()

        def step(i, _):
            pltpu.sync_copy(csrc, accc.at[ridx2.at[i]], add=True)
            return 0

        lax.fori_loop(0, nch, step, 0, unroll=False)
        plsc.subcore_barrier()
        pltpu.sync_copy(accc.at[pl.ds(sid * NPC, NPC)],
                        scnt_o.at[cid, pl.ds(sid * NPC, NPC)])

    return k(row_rs, zcnt, cnt_src)


def _edge1_body(pr_ref, pc_ref, ea_ref,
                w1cT_ref, b1_ref, w2T_ref, b2_ref,
                v1bT_ref, c1_ref, v2T_ref, c2_ref,
                e1_ref, h_ref, acc_ref):
    i = pl.program_id(0)

    @pl.when(i == 0)
    def _():
        acc_ref[...] = jnp.zeros_like(acc_ref)

    pr = pr_ref[:, 0:64]
    pc = pc_ref[...]
    ea = ea_ref[...]
    h1 = jnp.maximum(
        pr + pc[:, :64]
        + jnp.dot(ea, w1cT_ref[...], preferred_element_type=jnp.float32)
        + b1_ref[...], 0.0)
    e1 = jnp.dot(h1, w2T_ref[...], preferred_element_type=jnp.float32) + b2_ref[...]
    n1 = jnp.maximum(
        pc[:, 64:192]
        + jnp.dot(e1, v1bT_ref[...], preferred_element_type=jnp.float32)
        + c1_ref[...], 0.0)
    h = jnp.dot(n1, v2T_ref[...], preferred_element_type=jnp.float32) + c2_ref[...]
    e1_ref[...] = e1
    h_ref[...] = h

    gi = pr_ref[:, 64:65].astype(jnp.int32)
    oh = (gi == jax.lax.broadcasted_iota(jnp.int32, (EB, G), 1)
          ).astype(jnp.float32)
    ge1s = jax.lax.dot_general(oh, e1, (((0,), (0,)), ((), ())),
                               preferred_element_type=jnp.float32,
                               precision=jax.lax.Precision.HIGHEST)
    cg = jnp.sum(oh, axis=0)
    cgrow = jnp.concatenate([cg, jnp.zeros((64,), jnp.float32)])[None, :]
    se = jnp.sum(e1, axis=0)[None, :]
    sq = jnp.sum(e1 * e1, axis=0)[None, :]
    acc_ref[0:G, :] += ge1s
    acc_ref[G:G + 1, :] += cgrow
    acc_ref[G + 1:G + 2, :] += se
    acc_ref[G + 2:G + 3, :] += sq


def _edge1(prg, pcg, ea, w1cT, b1, w2T, b2, v1bT, c1, v2T, c2):
    EC = prg.shape[0]
    blk = lambda r, c: pl.BlockSpec((EB, c), lambda i: (i, 0))
    full = lambda shape: pl.BlockSpec(shape, lambda i: tuple(0 for _ in shape))
    return pl.pallas_call(
        _edge1_body,
        grid=(EC // EB,),
        in_specs=[blk(EB, 128), blk(EB, 256), blk(EB, 16),
                  full((16, 64)), full((1, 64)), full((64, 128)), full((1, 128)),
                  full((128, 128)), full((1, 128)), full((128, 128)), full((1, 128))],
        out_specs=[blk(EB, 128), blk(EB, 128),
                   pl.BlockSpec((72, 128), lambda i: (0, 0))],
        out_shape=(jax.ShapeDtypeStruct((EC, 128), jnp.float32),
                   jax.ShapeDtypeStruct((EC, 128), jnp.float32),
                   jax.ShapeDtypeStruct((72, 128), jnp.float32)),
        interpret=_INTERP,
    )(prg, pcg, ea, w1cT, b1, w2T, b2, v1bT, c1, v2T, c2)


def _node1_body(sa_ref, sb_ref, c_ref, x_ref,
                z1aT_ref, z1bT_ref, d1_ref, z2T_ref, d2_ref,
                x1_ref, st_ref):
    i = pl.program_id(0)

    @pl.when(i == 0)
    def _():
        st_ref[...] = jnp.zeros_like(st_ref)

    s = sa_ref[0] + sa_ref[1] + sb_ref[0] + sb_ref[1]
    c = c_ref[0, :, 0:1] + c_ref[1, :, 0:1]
    agg = s / jnp.maximum(c, 1.0)
    xh = jnp.maximum(
        jnp.dot(x_ref[...], z1aT_ref[...], preferred_element_type=jnp.float32)
        + jnp.dot(agg, z1bT_ref[...], preferred_element_type=jnp.float32)
        + d1_ref[...], 0.0)
    x1 = jnp.dot(xh, z2T_ref[...], preferred_element_type=jnp.float32) + d2_ref[...]
    x1_ref[...] = x1
    st_ref[0:1, :] += jnp.sum(x1, axis=0)[None, :]
    st_ref[1:2, :] += jnp.sum(x1 * x1, axis=0)[None, :]


def _node1(ssA, ssB, scnt, x, z1aT, z1bT, d1, z2T, d2):
    return pl.pallas_call(
        _node1_body,
        grid=(N // NB,),
        in_specs=[pl.BlockSpec((2, NB, 128), lambda i: (0, i, 0)),
                  pl.BlockSpec((2, NB, 128), lambda i: (0, i, 0)),
                  pl.BlockSpec((2, NB, 128), lambda i: (0, i, 0)),
                  pl.BlockSpec((NB, 128), lambda i: (i, 0)),
                  pl.BlockSpec((128, 128), lambda i: (0, 0)),
                  pl.BlockSpec((128, 128), lambda i: (0, 0)),
                  pl.BlockSpec((1, 128), lambda i: (0, 0)),
                  pl.BlockSpec((128, 256), lambda i: (0, 0)),
                  pl.BlockSpec((1, 256), lambda i: (0, 0))],
        out_specs=[pl.BlockSpec((NB, 256), lambda i: (i, 0)),
                   pl.BlockSpec((8, 256), lambda i: (0, 0))],
        out_shape=(jax.ShapeDtypeStruct((N, 256), jnp.float32),
                   jax.ShapeDtypeStruct((8, 256), jnp.float32)),
        interpret=_INTERP,
    )(ssA, ssB, scnt, x, z1aT, z1bT, d1, z2T, d2)


def _node2_body(x1_ref, st_ref, gn_ref, bn_ref, u1aT_ref, u1bT_ref,
                qr_ref, qc_ref):
    m = st_ref[0:1, :] * (1.0 / N)
    v = st_ref[1:2, :] * (1.0 / N) - m * m
    x1b = (x1_ref[...] - m) * jax.lax.rsqrt(v + 1e-5) * gn_ref[...] + bn_ref[...]
    qr_ref[...] = jnp.dot(x1b, u1aT_ref[...], preferred_element_type=jnp.float32)
    qc_ref[...] = jnp.dot(x1b, u1bT_ref[...], preferred_element_type=jnp.float32)


def _node2(x1, st, gn, bn, u1aT, u1bT):
    return pl.pallas_call(
        _node2_body,
        grid=(N // NB,),
        in_specs=[pl.BlockSpec((NB, 256), lambda i: (i, 0)),
                  pl.BlockSpec((8, 256), lambda i: (0, 0)),
                  pl.BlockSpec((1, 256), lambda i: (0, 0)),
                  pl.BlockSpec((1, 256), lambda i: (0, 0)),
                  pl.BlockSpec((256, 128), lambda i: (0, 0)),
                  pl.BlockSpec((256, 128), lambda i: (0, 0))],
        out_specs=[pl.BlockSpec((NB, 128), lambda i: (i, 0)),
                   pl.BlockSpec((NB, 128), lambda i: (i, 0))],
        out_shape=(jax.ShapeDtypeStruct((N, 128), jnp.float32),
                   jax.ShapeDtypeStruct((N, 128), jnp.float32)),
        interpret=_INTERP,
    )(x1, st, gn, bn, u1aT, u1bT)


def _edge2_body(qr_ref, qc_ref, e1_ref, g_ref, est_ref,
                ge_ref, be_ref, u1cT_ref, e1b_ref, u2T_ref, e2b_ref,
                acc_ref):
    i = pl.program_id(0)

    @pl.when(i == 0)
    def _():
        acc_ref[...] = jnp.zeros_like(acc_ref)

    mean = est_ref[0:1, :] * (1.0 / E)
    var = est_ref[1:2, :] * (1.0 / E) - mean * mean
    a = ge_ref[...] * jax.lax.rsqrt(var + 1e-5)
    bb = be_ref[...] - mean * a
    e1bn = e1_ref[...] * a + bb
    h2 = jnp.maximum(
        qr_ref[...] + qc_ref[...]
        + jnp.dot(e1bn, u1cT_ref[...], preferred_element_type=jnp.float32)
        + e1b_ref[...], 0.0)
    e2 = jnp.dot(h2, u2T_ref[...], preferred_element_type=jnp.float32) + e2b_ref[...]
    gi = g_ref[:, 64:65].astype(jnp.int32)
    oh = (gi == jax.lax.broadcasted_iota(jnp.int32, (EB, G), 1)
          ).astype(jnp.float32)
    acc_ref[...] += jax.lax.dot_general(oh, e2, (((0,), (0,)), ((), ())),
                                        preferred_element_type=jnp.float32,
                                        precision=jax.lax.Precision.HIGHEST)


def _edge2(qrg, qcg, e1, gs, est, ge, be, u1cT, e1b, u2T, e2b):
    EC = qrg.shape[0]
    blk = lambda c: pl.BlockSpec((EB, c), lambda i: (i, 0))
    full = lambda shape: pl.BlockSpec(shape, lambda i: tuple(0 for _ in shape))
    return pl.pallas_call(
        _edge2_body,
        grid=(EC // EB,),
        in_specs=[blk(128), blk(128), blk(128),
                  pl.BlockSpec((EB, 128), lambda i: (i, 0)),
                  full((8, 128)), full((1, 128)), full((1, 128)),
                  full((128, 128)), full((1, 128)), full((128, 128)), full((1, 128))],
        out_specs=pl.BlockSpec((G, 128), lambda i: (0, 0)),
        out_shape=jax.ShapeDtypeStruct((G, 128), jnp.float32),
        interpret=_INTERP,
    )(qrg, qcg, e1, gs, est, ge, be, u1cT, e1b, u2T, e2b)


def _glob_body(acc1_ref, acc2_ref,
               g1geT_ref, gb1_ref, g2T_ref, gb2_ref,
               h1uT_ref, h1geT_ref, hb1_ref, h2T_ref, hb2_ref,
               out_ref):
    inv = 1.0 / jnp.maximum(acc1_ref[G:G + 1, 0:G], 1.0)
    eye = (jax.lax.broadcasted_iota(jnp.int32, (G, G), 0)
           == jax.lax.broadcasted_iota(jnp.int32, (G, G), 1)).astype(jnp.float32)
    dinv = eye * inv
    ge1 = jnp.dot(dinv, acc1_ref[0:G, :], preferred_element_type=jnp.float32,
                  precision=jax.lax.Precision.HIGHEST)
    u1h = jnp.maximum(
        jnp.dot(ge1, g1geT_ref[...], preferred_element_type=jnp.float32)
        + gb1_ref[...], 0.0)
    u1 = jnp.dot(u1h, g2T_ref[...], preferred_element_type=jnp.float32) + gb2_ref[...]
    ge2 = jnp.dot(dinv, acc2_ref[...], preferred_element_type=jnp.float32,
                  precision=jax.lax.Precision.HIGHEST)
    u2h = jnp.maximum(
        jnp.dot(u1, h1uT_ref[...], preferred_element_type=jnp.float32)
        + jnp.dot(ge2, h1geT_ref[...], preferred_element_type=jnp.float32)
        + hb1_ref[...], 0.0)
    out_ref[...] = (jnp.dot(u2h, h2T_ref[...], preferred_element_type=jnp.float32)
                    + hb2_ref[...])


def _glob(acc1, acc2, g1geT, gb1, g2T, gb2, h1uT, h1geT, hb1, h2T, hb2):
    return pl.pallas_call(
        _glob_body,
        out_shape=jax.ShapeDtypeStruct((G, 1), jnp.float32),
        interpret=_INTERP,
    )(acc1, acc2, g1geT, gb1, g2T, gb2, h1uT, h1geT, hb1, h2T, hb2)


def kernel(x, edge_index, edge_attr, batch, params):
    row = edge_index[0]
    col = edge_index[1]
    p1 = params['l1']
    (W1, b1), (W2, b2) = p1['edge']
    (V1, c1), (V2, c2) = p1['node1']
    (Z1, d1), (Z2, d2) = p1['node2']
    (Gw1, gb1), (Gw2, gb2) = p1['glob']
    gme, gbe = params['bn_edge']
    gmn, gbn = params['bn_node']
    p2 = params['l2']
    (U1, e1b), (U2, e2b) = p2['edge']
    (Hw1, hb1), (Hw2, hb2) = p2['glob']

    r2 = lambda v: v[None, :]

    wrT = W1[:, :128].T
    wcTp = jnp.concatenate([W1[:, 128:256].T, V1[:, :128].T,
                            jnp.zeros((128, 64), jnp.float32)], axis=1)
    tr, tcol = _proj(x, batch[:, None], wrT, wcTp)

    def chunk_idx(a, ci):
        nch = CH_NCH[ci]
        off = CH_OFF[ci]
        return a[off:off + NW * nch * CB].reshape(NW, nch, CB)

    row_c = [chunk_idx(row, i) for i in range(NCHUNK)]
    col_c = [chunk_idx(col, i) for i in range(NCHUNK)]
    bnd = CH_OFF + (E,)
    ea_c = [edge_attr[bnd[i]:bnd[i + 1]] for i in range(NCHUNK)]

    zsum = jnp.zeros((NPAD, 128), jnp.float32)
    cnt_src = jnp.zeros((CB, 128), jnp.float32).at[:, 0].set(1.0)
    row_rs_full = row.reshape(NW, E // NW // CB, CB)
    scnt = _sc_count(row_rs_full, zsum, cnt_src)

    we = (W1[:, 256:].T, r2(b1), W2.T, r2(b2), V1[:, 128:].T, r2(c1), V2.T, r2(c2))
    pg = [_sc_gather((tr, tcol), (128, 256), row_c[i], col_c[i], CH_NCH[i])
          for i in range(NCHUNK)]
    eo = [_edge1(pg[i][0], pg[i][1], ea_c[i], *we) for i in range(NCHUNK)]
    ss = [_sc_scatter(eo[i][1], row_c[i], zsum, CH_NCH[i]) for i in range(NCHUNK)]
    acc1 = sum(e[2] for e in eo)

    ssA = ss[0]
    ssB = ss[1] + ss[2]
    x1, xst = _node1(ssA, ssB, scnt, x,
                     Z1[:, :128].T, Z1[:, 128:].T, r2(d1), Z2.T, r2(d2))
    qrow, qcol = _node2(x1, xst, r2(gmn), r2(gbn),
                        U1[:, :256].T, U1[:, 256:512].T)

    est = jnp.zeros((8, 128), jnp.float32).at[0:2, :].set(acc1[G + 1:G + 3, :])
    w2e = (r2(gme), r2(gbe), U1[:, 512:].T, r2(e1b), U2.T, r2(e2b))
    qg = [_sc_gather((qrow, qcol), (128, 128), row_c[i], col_c[i], CH_NCH[i])
          for i in range(NCHUNK)]
    acc2 = sum(_edge2(qg[i][0], qg[i][1], eo[i][0], pg[i][0], est, *w2e)
               for i in range(NCHUNK))

    return _glob(acc1, acc2, Gw1[:, 1:].T, r2(gb1), Gw2.T, r2(gb2),
                 Hw1[:, :1].T, Hw1[:, 1:].T, r2(hb1), Hw2.T, r2(hb2))
